# baseline (speedup 1.0000x reference)
import functools

import jax
import jax.numpy as jnp
from jax import lax
from jax.experimental import pallas as pl
from jax.experimental.pallas import tpu as pltpu
from jax.experimental.pallas import tpu_sc as plsc

NTOKEN = 1000
BATCH, SEQ = 1024, 50
ROWS = BATCH * SEQ
NELEM = ROWS * NTOKEN
W128 = 128
RR = NELEM // W128
NUM_CORES, NUM_SUBCORES, LANES = 2, 16, 16
NW = NUM_CORES * NUM_SUBCORES
ROWS_PER_W = ROWS // NW
BLK = 800
NBLK = RR // BLK
FULL_ROUNDS = NBLK // NW
TAIL = NBLK - FULL_ROUNDS * NW


G = ROWS_PER_W // LANES


def _sc_scatter_body(x_hbm, buf_hbm, xv_ref, ones_ref, ssem):
    cid = lax.axis_index("c")
    sid = lax.axis_index("s")
    wid = sid * NUM_CORES + cid
    row_base = wid * ROWS_PER_W

    pltpu.sync_copy(x_hbm.at[pl.ds(row_base, ROWS_PER_W)], xv_ref)
    ones_ref[...] = jnp.ones((LANES,), jnp.float32)

    lane = lax.iota(jnp.int32, LANES)
    copies = []
    for g in range(G):
        xv = xv_ref[pl.ds(g * LANES, LANES)]
        col = jnp.clip(xv, 0, NTOKEN - 1)
        idx = (row_base + g * LANES + lane) * NTOKEN + col
        c = pltpu.make_async_copy(ones_ref, buf_hbm.at[idx], ssem)
        c.start()
        copies.append(c)
    for c in copies:
        c.wait()


_sc_scatter = pl.kernel(
    _sc_scatter_body,
    out_type=(),
    mesh=plsc.VectorSubcoreMesh(core_axis_name="c", subcore_axis_name="s"),
    scratch_types=[
        pltpu.VMEM((ROWS_PER_W,), jnp.int32),
        pltpu.VMEM((LANES,), jnp.float32),
        pltpu.SemaphoreType.DMA,
    ],
)


@jax.jit
def _indicator(x):
    buf = jax.new_ref(jnp.zeros((NELEM,), jnp.float32))
    _sc_scatter(x.reshape(ROWS), buf)
    return jax.freeze(buf).reshape(BATCH, SEQ, NTOKEN)


def kernel(x):
    return _indicator(x)


# speedup vs baseline: 1.5845x; 1.5845x over previous
import functools

import jax
import jax.numpy as jnp
from jax import lax
from jax.experimental import pallas as pl
from jax.experimental.pallas import tpu as pltpu
from jax.experimental.pallas import tpu_sc as plsc

NTOKEN = 1000
BATCH, SEQ = 1024, 50
ROWS = BATCH * SEQ
NUM_CORES, NUM_SUBCORES, LANES = 2, 16, 16
NW = NUM_CORES * NUM_SUBCORES
ROWS_PER_W = ROWS // NW
TB = 32
NB = ROWS_PER_W // TB


def _sc_body(x_hbm, out_hbm, xv_ref, bufa_ref, bufb_ref, sema, semb):
    cid = lax.axis_index("c")
    sid = lax.axis_index("s")
    wid = sid * NUM_CORES + cid
    row_base = pl.multiple_of(wid * ROWS_PER_W, ROWS_PER_W)

    pltpu.sync_copy(x_hbm.at[pl.ds(row_base, ROWS_PER_W)], xv_ref)

    bufs = (bufa_ref, bufb_ref)
    sems = (sema, semb)

    zeros16 = jnp.zeros((LANES,), jnp.float32)

    def zrow(r, carry):
        for buf in bufs:
            for o in range(62):
                buf[r, pl.ds(o * LANES, LANES)] = zeros16
            buf[r, pl.ds(NTOKEN - LANES, LANES)] = zeros16
        return carry

    lax.fori_loop(0, TB, zrow, 0)

    lane = lax.iota(jnp.int32, LANES)

    def put(buf, b, value):
        for g in range(TB // LANES):
            c16 = xv_ref[pl.ds(b * TB + g * LANES, LANES)]
            col = jnp.clip(c16, 0, NTOKEN - 1)
            for j in range(LANES):
                cj = col[j]
                raw = c16[j]
                win = pl.multiple_of((cj >> 4) << 4, LANES)
                sel = (cj & 15) | ((raw >> 31) & 16)
                one16 = jnp.where(
                    lane == sel, jnp.float32(value), jnp.float32(0.0))
                buf[g * LANES + j, pl.ds(win, LANES)] = one16

    def block_dma(buf, b, sem):
        dst = out_hbm.at[pl.ds(row_base + pl.multiple_of(b * TB, TB), TB), :]
        return pltpu.make_async_copy(buf, dst, sem)

    @pl.loop(0, NB, step=2)
    def body(b0):
        for k in range(2):
            b = b0 + k
            buf, sem = bufs[k], sems[k]

            @pl.when(b0 > 0)
            def _():
                block_dma(buf, b, sem).wait()
                put(buf, b - 2, 0.0)

            put(buf, b, 1.0)
            block_dma(buf, b, sem).start()

    for k in range(2):
        block_dma(bufs[k], NB - 2 + k, sems[k]).wait()


@jax.jit
def _indicator(x):
    run = pl.kernel(
        _sc_body,
        out_type=jax.ShapeDtypeStruct((ROWS, NTOKEN), jnp.float32),
        mesh=plsc.VectorSubcoreMesh(core_axis_name="c", subcore_axis_name="s"),
        scratch_types=[
            pltpu.VMEM((ROWS_PER_W,), jnp.int32),
            pltpu.VMEM((TB, NTOKEN), jnp.float32),
            pltpu.VMEM((TB, NTOKEN), jnp.float32),
            pltpu.SemaphoreType.DMA,
            pltpu.SemaphoreType.DMA,
        ],
    )
    out2d = run(x.reshape(ROWS))
    return out2d.reshape(BATCH, SEQ, NTOKEN)


def kernel(x):
    return _indicator(x)
